# baseline (speedup 1.0000x reference)
import functools

import jax
import jax.numpy as jnp
from jax import lax
from jax.experimental import pallas as pl
from jax.experimental.pallas import tpu as pltpu
from jax.experimental.pallas import tpu_sc as plsc

B = 4096
SEQ = 50
EMB = 128
HID = 2048
CLS = 1000

EMB2 = EMB // 2

NC = 2
NS = 16
NW = NC * NS
NGROUPS = SEQ
NBUF = 4
CHUNKS = 2
BCH = B // CHUNKS
GR = BCH // NW


def _gather_body(idx_hbm, table_hbm, out_hbm, idx_v, rows_v, gsems, wsems):
    wid = lax.axis_index("s") * NC + lax.axis_index("c")
    row0 = wid * GR
    pltpu.sync_copy(idx_hbm.at[wid], idx_v)

    def g(j):
        return pltpu.make_async_copy(table_hbm.at[idx_v.at[j]],
                                     rows_v.at[j % NBUF], gsems.at[j % NBUF])

    def w(j):
        return pltpu.make_async_copy(
            rows_v.at[j % NBUF],
            out_hbm.at[pl.ds(row0, GR), pl.ds(j * EMB, EMB)],
            wsems.at[j % NBUF])

    for j0 in range(NBUF - 1):
        g(j0).start()

    def body(j, _):
        g(j).wait()
        w(j).start()
        nj = j + NBUF - 1

        @pl.when(nj < NGROUPS)
        def _():
            @pl.when(j >= 1)
            def _():
                w(j - 1).wait()

            g(nj).start()

        return 0

    lax.fori_loop(0, NGROUPS, body, 0)
    for t in range(NBUF):
        w(NGROUPS - NBUF + t).wait()


def _sc_gather(xf, table):
    mesh = plsc.VectorSubcoreMesh(core_axis_name="c", subcore_axis_name="s")
    return pl.kernel(
        _gather_body,
        mesh=mesh,
        out_type=jax.ShapeDtypeStruct((BCH, SEQ * EMB), jnp.float32),
        scratch_types=[
            pltpu.VMEM((NGROUPS, GR), jnp.int32),
            pltpu.VMEM((NBUF, GR, EMB), jnp.float32),
            pltpu.SemaphoreType.DMA((NBUF,)),
            pltpu.SemaphoreType.DMA((NBUF,)),
        ],
    )(xf, table)


BM = 256
KTOT = SEQ * EMB


def _mlp_body(flat_ref, w1_ref, b1_ref, w2_ref, b2_ref, out_ref):
    h = jnp.dot(flat_ref[...].astype(jnp.bfloat16), w1_ref[...],
                preferred_element_type=jnp.float32) + b1_ref[...]
    out_ref[...] = (
        jnp.dot(h.astype(jnp.bfloat16), w2_ref[...],
                preferred_element_type=jnp.float32)
        + b2_ref[...]
    )


def _tc_mlp(flat, W1, b1, W2, b2):
    return pl.pallas_call(
        _mlp_body,
        grid=(BCH // BM,),
        in_specs=[
            pl.BlockSpec((BM, KTOT), lambda m: (m, 0)),
            pl.BlockSpec((KTOT, HID), lambda m: (0, 0)),
            pl.BlockSpec((1, HID), lambda m: (0, 0)),
            pl.BlockSpec((HID, CLS), lambda m: (0, 0)),
            pl.BlockSpec((1, CLS), lambda m: (0, 0)),
        ],
        out_specs=pl.BlockSpec((BM, CLS), lambda m: (m, 0)),
        out_shape=jax.ShapeDtypeStruct((BCH, CLS), jnp.float32),
        compiler_params=pltpu.CompilerParams(
            dimension_semantics=("arbitrary",),
        ),
    )(flat, W1, b1.reshape(1, HID), W2, b2.reshape(1, CLS))


def kernel(x, embed_table, W1, b1, W2, b2):
    xt = x.astype(jnp.int32).reshape(CHUNKS, NW, GR, SEQ).transpose(0, 1, 3, 2)
    w1b = W1.astype(jnp.bfloat16)
    w2b = W2.astype(jnp.bfloat16)
    flats = [_sc_gather(xt[c], embed_table) for c in range(CHUNKS)]
    outs = [_tc_mlp(f, w1b, b1, w2b, b2) for f in flats]
    return jnp.concatenate(outs, axis=0)


# speedup vs baseline: 1.0480x; 1.0480x over previous
import jax
import jax.numpy as jnp
from jax import lax
from jax.experimental import pallas as pl
from jax.experimental.pallas import tpu as pltpu
from jax.experimental.pallas import tpu_sc as plsc

B = 4096
SEQ = 50
EMB = 128
HID = 2048
CLS = 1000

NC = 2
NS = 16
NW = NC * NS
GR = B // NW
NGROUPS = SEQ
NBUF = 6


def _gather_body(idx_hbm, table_hbm, out_hbm, idx_v, rows_v, gsems, wsems):
    wid = lax.axis_index("s") * NC + lax.axis_index("c")
    row0 = wid * GR
    pltpu.sync_copy(idx_hbm.at[wid], idx_v)

    def g(j):
        return pltpu.make_async_copy(table_hbm.at[idx_v.at[j]],
                                     rows_v.at[j % NBUF], gsems.at[j % NBUF])

    def w(j):
        return pltpu.make_async_copy(
            rows_v.at[j % NBUF],
            out_hbm.at[pl.ds(row0, GR), pl.ds(j * EMB, EMB)],
            wsems.at[j % NBUF])

    for j0 in range(NBUF - 1):
        g(j0).start()

    def body(j, _):
        g(j).wait()
        w(j).start()
        nj = j + NBUF - 1

        @pl.when(nj < NGROUPS)
        def _():
            @pl.when(j >= 1)
            def _():
                w(j - 1).wait()

            g(nj).start()

        return 0

    lax.fori_loop(0, NGROUPS, body, 0)
    for t in range(NBUF):
        w(NGROUPS - NBUF + t).wait()


def _sc_gather(xf, table):
    mesh = plsc.VectorSubcoreMesh(core_axis_name="c", subcore_axis_name="s")
    return pl.kernel(
        _gather_body,
        mesh=mesh,
        out_type=jax.ShapeDtypeStruct((B, SEQ * EMB), jnp.float32),
        scratch_types=[
            pltpu.VMEM((NGROUPS, GR), jnp.int32),
            pltpu.VMEM((NBUF, GR, EMB), jnp.float32),
            pltpu.SemaphoreType.DMA((NBUF,)),
            pltpu.SemaphoreType.DMA((NBUF,)),
        ],
    )(xf, table)


BM = 256
KTOT = SEQ * EMB


def _mlp_body(flat_ref, w1_ref, b1_ref, w2_ref, b2_ref, out_ref):
    h = jnp.dot(flat_ref[...].astype(jnp.bfloat16), w1_ref[...],
                preferred_element_type=jnp.float32) + b1_ref[...]
    out_ref[...] = (
        jnp.dot(h.astype(jnp.bfloat16), w2_ref[...],
                preferred_element_type=jnp.float32)
        + b2_ref[...]
    )


def _tc_mlp(flat, W1, b1, W2, b2):
    return pl.pallas_call(
        _mlp_body,
        grid=(B // BM,),
        in_specs=[
            pl.BlockSpec((BM, KTOT), lambda m: (m, 0)),
            pl.BlockSpec((KTOT, HID), lambda m: (0, 0)),
            pl.BlockSpec((1, HID), lambda m: (0, 0)),
            pl.BlockSpec((HID, CLS), lambda m: (0, 0)),
            pl.BlockSpec((1, CLS), lambda m: (0, 0)),
        ],
        out_specs=pl.BlockSpec((BM, CLS), lambda m: (m, 0)),
        out_shape=jax.ShapeDtypeStruct((B, CLS), jnp.float32),
        compiler_params=pltpu.CompilerParams(
            dimension_semantics=("arbitrary",),
        ),
    )(flat, W1, b1.reshape(1, HID), W2, b2.reshape(1, CLS))


def kernel(x, embed_table, W1, b1, W2, b2):
    xt = x.astype(jnp.int32).reshape(NW, GR, SEQ).transpose(0, 2, 1)
    flat = _sc_gather(xt, embed_table)
    return _tc_mlp(flat, W1.astype(jnp.bfloat16), b1,
                   W2.astype(jnp.bfloat16), b2)
